# baseline (speedup 1.0000x reference)
import jax
import jax.numpy as jnp
from jax import lax
from jax.experimental import pallas as pl
from jax.experimental.pallas import tpu as pltpu
from jax.experimental.pallas import tpu_sc as plsc

B = 4096
F = 26
D = 64
BF = B * F
NC, NS = 2, 16
NW = NC * NS
PER_W = BF // NW
CHUNK = 64
NCH = PER_W // CHUNK
NROUNDS = NCH // 2
LANES = 16
FIELD_SIZE = 100000
GW = 8 * D


def _body(x_hbm, w_hbm, out_hbm, idx_v, tv, buf0, buf1, ob0, ob1,
          gs0, gs1, cs0, cs1):
    wid = lax.axis_index("s") * NC + lax.axis_index("c")
    base = wid * PER_W
    pltpu.sync_copy(x_hbm.at[pl.ds(base, PER_W)], idx_v)

    def off(t, carry):
        pos = base + t * LANES + lax.iota(jnp.int32, LANES)
        sl = pl.ds(t * LANES, LANES)
        v = idx_v[sl] + lax.rem(pos, F) * FIELD_SIZE
        idx_v[sl] = v
        tv[sl] = lax.shift_right_logical(v, 3)
        return carry

    lax.fori_loop(0, PER_W // LANES, off, 0)

    bufs = (buf0, buf1)
    obufs = (ob0, ob1)
    gsems = (gs0, gs1)
    csems = (cs0, cs1)

    def gather_desc(j, b):
        return pltpu.make_async_copy(
            w_hbm.at[tv.at[pl.ds(j * CHUNK, CHUNK)]], bufs[b], gsems[b])

    def copy_desc(j, b):
        return pltpu.make_async_copy(
            obufs[b],
            out_hbm.at[pl.ds(wid * (PER_W // 2) + j * (CHUNK // 2),
                             CHUNK // 2)],
            csems[b])

    def select(j, b):
        for g in range(CHUNK // LANES):
            hv = idx_v[pl.ds(j * CHUNK + g * LANES, LANES)]
            for l in range(LANES):
                roff = (hv[l] & 7) * D
                srow = g * LANES + l
                orow = g * (LANES // 2) + (l >> 1)
                ocol = (l & 1) * D
                for t in range(D // LANES):
                    obufs[b][orow, pl.ds(ocol + t * LANES, LANES)] = (
                        bufs[b][srow, pl.ds(roff + t * LANES, LANES)])

    gather_desc(0, 0).start()
    gather_desc(1, 1).start()

    def rnd(k, carry):
        for b in range(2):
            j = 2 * k + b
            gather_desc(j, b).wait()

            @pl.when(k > 0)
            def _():
                copy_desc(j - 2, b).wait()

            select(j, b)
            copy_desc(j, b).start()

            @pl.when(k < NROUNDS - 1)
            def _():
                gather_desc(j + 2, b).start()

        return carry

    lax.fori_loop(0, NROUNDS, rnd, 0)
    copy_desc(NCH - 2, 0).wait()
    copy_desc(NCH - 1, 1).wait()


def kernel(x, weight):
    mesh = plsc.VectorSubcoreMesh(
        core_axis_name="c", subcore_axis_name="s",
        num_cores=NC, num_subcores=NS,
    )
    lookup = pl.kernel(
        _body,
        out_type=jax.ShapeDtypeStruct((BF // 2, 2 * D), jnp.float32),
        mesh=mesh,
        scratch_types=[
            pltpu.VMEM((PER_W,), jnp.int32),
            pltpu.VMEM((PER_W,), jnp.int32),
            pltpu.VMEM((CHUNK, GW), jnp.float32),
            pltpu.VMEM((CHUNK, GW), jnp.float32),
            pltpu.VMEM((CHUNK // 2, 2 * D), jnp.float32),
            pltpu.VMEM((CHUNK // 2, 2 * D), jnp.float32),
            pltpu.SemaphoreType.DMA,
            pltpu.SemaphoreType.DMA,
            pltpu.SemaphoreType.DMA,
            pltpu.SemaphoreType.DMA,
        ],
    )
    out = lookup(x.reshape(BF), weight.reshape(weight.shape[0] // 8, GW))
    return out.reshape(B, F * D)


# speedup vs baseline: 2.2539x; 2.2539x over previous
import jax
import jax.numpy as jnp
from jax import lax
from jax.experimental import pallas as pl
from jax.experimental.pallas import tpu as pltpu
from jax.experimental.pallas import tpu_sc as plsc

B = 4096
F = 26
D = 64
BF = B * F
NC, NS = 2, 16
NW = NC * NS
PER_W = BF // NW
CHUNK = 16
NCH = PER_W // CHUNK
NBUF = 4
NROUNDS = NCH // NBUF
LANES = 16
FIELD_SIZE = 100000


def _body(x_hbm, w_hbm, out_hbm, idx_v, tv, buf0, buf1, buf2, buf3,
          ob0, ob1, ob2, ob3, gs0, gs1, gs2, gs3, cs0, cs1, cs2, cs3):
    wid = lax.axis_index("s") * NC + lax.axis_index("c")
    base = wid * PER_W
    pltpu.sync_copy(x_hbm.at[pl.ds(base, PER_W)], idx_v)

    def off(t, carry):
        pos = base + t * LANES + lax.iota(jnp.int32, LANES)
        sl = pl.ds(t * LANES, LANES)
        v = idx_v[sl] + lax.rem(pos, F) * FIELD_SIZE
        idx_v[sl] = v
        tv[sl] = lax.shift_right_logical(v, 3)
        return carry

    lax.fori_loop(0, PER_W // LANES, off, 0)

    bufs = (buf0, buf1, buf2, buf3)
    obufs = (ob0, ob1, ob2, ob3)
    gsems = (gs0, gs1, gs2, gs3)
    csems = (cs0, cs1, cs2, cs3)

    def fire_chunk(j, b):
        for g in range(CHUNK // LANES):
            vec = tv[pl.ds(j * CHUNK + g * LANES, LANES)]
            for l in range(LANES):
                tile = vec[l]
                pltpu.async_copy(
                    w_hbm.at[pl.ds(tile, 1)],
                    bufs[b].at[pl.ds(g * LANES + l, 1)], gsems[b])

    def gather_drain(b):
        pltpu.make_async_copy(
            w_hbm.at[pl.ds(0, CHUNK)], bufs[b], gsems[b]).wait()

    def copy_desc(j, b):
        return pltpu.make_async_copy(
            obufs[b],
            out_hbm.at[pl.ds(wid * (PER_W // 2) + j * (CHUNK // 2),
                             CHUNK // 2)],
            csems[b])

    def select(j, b):
        for g in range(CHUNK // LANES):
            hv = idx_v[pl.ds(j * CHUNK + g * LANES, LANES)]
            for l in range(LANES):
                rit = hv[l] & 7
                srow = g * LANES + l
                orow = g * (LANES // 2) + (l >> 1)
                ocol = (l & 1) * D
                for t in range(D // LANES):
                    obufs[b][orow, pl.ds(ocol + t * LANES, LANES)] = (
                        bufs[b][srow, rit, pl.ds(t * LANES, LANES)])

    for b in range(NBUF):
        fire_chunk(b, b)

    def rnd(k, carry):
        for b in range(NBUF):
            j = NBUF * k + b
            gather_drain(b)

            @pl.when(k > 0)
            def _():
                copy_desc(j - NBUF, b).wait()

            select(j, b)
            copy_desc(j, b).start()

            @pl.when(k < NROUNDS - 1)
            def _():
                fire_chunk(j + NBUF, b)

        return carry

    lax.fori_loop(0, NROUNDS, rnd, 0)
    for b in range(NBUF):
        copy_desc(NCH - NBUF + b, b).wait()


def kernel(x, weight):
    mesh = plsc.VectorSubcoreMesh(
        core_axis_name="c", subcore_axis_name="s",
        num_cores=NC, num_subcores=NS,
    )
    lookup = pl.kernel(
        _body,
        out_type=jax.ShapeDtypeStruct((BF // 2, 2 * D), jnp.float32),
        mesh=mesh,
        scratch_types=[
            pltpu.VMEM((PER_W,), jnp.int32),
            pltpu.VMEM((PER_W,), jnp.int32),
            pltpu.VMEM((CHUNK, 8, D), jnp.float32),
            pltpu.VMEM((CHUNK, 8, D), jnp.float32),
            pltpu.VMEM((CHUNK, 8, D), jnp.float32),
            pltpu.VMEM((CHUNK, 8, D), jnp.float32),
            pltpu.VMEM((CHUNK // 2, 2 * D), jnp.float32),
            pltpu.VMEM((CHUNK // 2, 2 * D), jnp.float32),
            pltpu.VMEM((CHUNK // 2, 2 * D), jnp.float32),
            pltpu.VMEM((CHUNK // 2, 2 * D), jnp.float32),
            pltpu.SemaphoreType.DMA,
            pltpu.SemaphoreType.DMA,
            pltpu.SemaphoreType.DMA,
            pltpu.SemaphoreType.DMA,
            pltpu.SemaphoreType.DMA,
            pltpu.SemaphoreType.DMA,
            pltpu.SemaphoreType.DMA,
            pltpu.SemaphoreType.DMA,
        ],
    )
    out = lookup(x.reshape(BF), weight.reshape(weight.shape[0] // 8, 8, D))
    return out.reshape(B, F * D)


# speedup vs baseline: 2.2549x; 1.0004x over previous
import jax
import jax.numpy as jnp
from jax import lax
from jax.experimental import pallas as pl
from jax.experimental.pallas import tpu as pltpu
from jax.experimental.pallas import tpu_sc as plsc

B = 4096
F = 26
D = 64
BF = B * F
NC, NS = 2, 16
NW = NC * NS
PER_W = BF // NW
CHUNK = 16
NCH = PER_W // CHUNK
NBUF = 4
NROUNDS = NCH // NBUF
LANES = 16
FIELD_SIZE = 100000


def _body(x_hbm, w_hbm, out_hbm, idx_r, idx_v, buf0, buf1, buf2, buf3,
          ob0, ob1, ob2, ob3, gs0, gs1, gs2, gs3, cs0, cs1, cs2, cs3):
    wid = lax.axis_index("s") * NC + lax.axis_index("c")
    base = wid * PER_W
    pltpu.sync_copy(x_hbm.at[pl.ds(base, PER_W)], idx_r)

    bufs = (buf0, buf1, buf2, buf3)
    obufs = (ob0, ob1, ob2, ob3)
    gsems = (gs0, gs1, gs2, gs3)
    csems = (cs0, cs1, cs2, cs3)

    def fire_chunk(j, b):
        for g in range(CHUNK // LANES):
            sl = pl.ds(j * CHUNK + g * LANES, LANES)
            pos = base + j * CHUNK + g * LANES + lax.iota(jnp.int32, LANES)
            vec = idx_r[sl] + lax.rem(pos, F) * FIELD_SIZE
            idx_v[sl] = vec
            for l in range(LANES):
                tile = lax.shift_right_logical(vec[l], 3)
                pltpu.async_copy(
                    w_hbm.at[pl.ds(tile, 1)],
                    bufs[b].at[pl.ds(g * LANES + l, 1)], gsems[b])

    def gather_drain(b):
        pltpu.make_async_copy(
            w_hbm.at[pl.ds(0, CHUNK)], bufs[b], gsems[b]).wait()

    def copy_desc(j, b):
        return pltpu.make_async_copy(
            obufs[b],
            out_hbm.at[pl.ds(wid * (PER_W // 2) + j * (CHUNK // 2),
                             CHUNK // 2)],
            csems[b])

    def select(j, b):
        for g in range(CHUNK // LANES):
            hv = idx_v[pl.ds(j * CHUNK + g * LANES, LANES)]
            for l in range(LANES):
                rit = hv[l] & 7
                srow = g * LANES + l
                orow = g * (LANES // 2) + (l >> 1)
                ocol = (l & 1) * D
                for t in range(D // LANES):
                    obufs[b][orow, pl.ds(ocol + t * LANES, LANES)] = (
                        bufs[b][srow, rit, pl.ds(t * LANES, LANES)])

    for b in range(NBUF):
        fire_chunk(b, b)

    def rnd(k, carry):
        for b in range(NBUF):
            j = NBUF * k + b
            gather_drain(b)

            @pl.when(k > 0)
            def _():
                copy_desc(j - NBUF, b).wait()

            select(j, b)
            copy_desc(j, b).start()

            @pl.when(k < NROUNDS - 1)
            def _():
                fire_chunk(j + NBUF, b)

        return carry

    lax.fori_loop(0, NROUNDS, rnd, 0)
    for b in range(NBUF):
        copy_desc(NCH - NBUF + b, b).wait()


def kernel(x, weight):
    mesh = plsc.VectorSubcoreMesh(
        core_axis_name="c", subcore_axis_name="s",
        num_cores=NC, num_subcores=NS,
    )
    lookup = pl.kernel(
        _body,
        out_type=jax.ShapeDtypeStruct((BF // 2, 2 * D), jnp.float32),
        mesh=mesh,
        scratch_types=[
            pltpu.VMEM((PER_W,), jnp.int32),
            pltpu.VMEM((PER_W,), jnp.int32),
            pltpu.VMEM((CHUNK, 8, D), jnp.float32),
            pltpu.VMEM((CHUNK, 8, D), jnp.float32),
            pltpu.VMEM((CHUNK, 8, D), jnp.float32),
            pltpu.VMEM((CHUNK, 8, D), jnp.float32),
            pltpu.VMEM((CHUNK // 2, 2 * D), jnp.float32),
            pltpu.VMEM((CHUNK // 2, 2 * D), jnp.float32),
            pltpu.VMEM((CHUNK // 2, 2 * D), jnp.float32),
            pltpu.VMEM((CHUNK // 2, 2 * D), jnp.float32),
            pltpu.SemaphoreType.DMA,
            pltpu.SemaphoreType.DMA,
            pltpu.SemaphoreType.DMA,
            pltpu.SemaphoreType.DMA,
            pltpu.SemaphoreType.DMA,
            pltpu.SemaphoreType.DMA,
            pltpu.SemaphoreType.DMA,
            pltpu.SemaphoreType.DMA,
        ],
    )
    out = lookup(x.reshape(BF), weight.reshape(weight.shape[0] // 8, 8, D))
    return out.reshape(B, F * D)


# speedup vs baseline: 2.3439x; 1.0395x over previous
import jax
import jax.numpy as jnp
from jax import lax
from jax.experimental import pallas as pl
from jax.experimental.pallas import tpu as pltpu
from jax.experimental.pallas import tpu_sc as plsc

B = 4096
F = 26
D = 64
BF = B * F
NC, NS = 2, 16
NW = NC * NS
PER_W = BF // NW
ROWS_W = B // NW
CHUNK = 16
NTC = (F * D) // 128
NCH = PER_W // CHUNK
NBUF = 4
NROUNDS = NCH // NBUF
LANES = 16
FIELD_SIZE = 100000


def _body(x_hbm, w_hbm, out_hbm, idx_r, idx_v, buf0, buf1, buf2, buf3,
          ob0, ob1, ob2, ob3, gs0, gs1, gs2, gs3, cs0, cs1, cs2, cs3):
    wid = lax.axis_index("s") * NC + lax.axis_index("c")
    base = wid * PER_W
    pltpu.sync_copy(x_hbm.at[pl.ds(base, PER_W)], idx_r)

    lane = lax.iota(jnp.int32, LANES)
    cvec = F * lax.shift_right_logical(lane, 1) + (lane & 1)

    bufs = (buf0, buf1, buf2, buf3)
    obufs = (ob0, ob1, ob2, ob3)
    gsems = (gs0, gs1, gs2, gs3)
    csems = (cs0, cs1, cs2, cs3)

    def fire_chunk(j, b):
        tr = lax.div(j, NTC)
        tc = lax.rem(j, NTC)
        p0 = tr * (8 * F) + 2 * tc
        vraw = plsc.load_gather(idx_r, [p0 + cvec])
        vec = vraw + (2 * tc + (lane & 1)) * FIELD_SIZE
        idx_v[pl.ds(j * LANES, LANES)] = vec
        for l in range(LANES):
            tile = lax.shift_right_logical(vec[l], 3)
            pltpu.async_copy(
                w_hbm.at[pl.ds(tile, 1)],
                bufs[b].at[pl.ds(l, 1)], gsems[b])

    def gather_drain(b):
        pltpu.make_async_copy(
            w_hbm.at[pl.ds(0, CHUNK)], bufs[b], gsems[b]).wait()

    def copy_desc(j, b):
        tr = lax.div(j, NTC)
        tc = lax.rem(j, NTC)
        row0 = pl.multiple_of(wid * ROWS_W + tr * 8, 8)
        col0 = pl.multiple_of(tc * 128, 128)
        return pltpu.make_async_copy(
            obufs[b], out_hbm.at[pl.ds(row0, 8), pl.ds(col0, 128)],
            csems[b])

    def select(j, b):
        hv = idx_v[pl.ds(j * LANES, LANES)]
        for l in range(LANES):
            rit = hv[l] & 7
            orow = l >> 1
            ocol = (l & 1) * D
            for t in range(D // LANES):
                obufs[b][orow, pl.ds(ocol + t * LANES, LANES)] = (
                    bufs[b][l, rit, pl.ds(t * LANES, LANES)])

    for b in range(NBUF):
        fire_chunk(b, b)

    def rnd(k, carry):
        for b in range(NBUF):
            j = NBUF * k + b
            gather_drain(b)

            @pl.when(k > 0)
            def _():
                copy_desc(j - NBUF, b).wait()

            select(j, b)
            copy_desc(j, b).start()

            @pl.when(k < NROUNDS - 1)
            def _():
                fire_chunk(j + NBUF, b)

        return carry

    lax.fori_loop(0, NROUNDS, rnd, 0)
    for b in range(NBUF):
        copy_desc(NCH - NBUF + b, b).wait()


def kernel(x, weight):
    mesh = plsc.VectorSubcoreMesh(
        core_axis_name="c", subcore_axis_name="s",
        num_cores=NC, num_subcores=NS,
    )
    lookup = pl.kernel(
        _body,
        out_type=jax.ShapeDtypeStruct((B, F * D), jnp.float32),
        mesh=mesh,
        scratch_types=[
            pltpu.VMEM((PER_W,), jnp.int32),
            pltpu.VMEM((PER_W,), jnp.int32),
            pltpu.VMEM((CHUNK, 8, D), jnp.float32),
            pltpu.VMEM((CHUNK, 8, D), jnp.float32),
            pltpu.VMEM((CHUNK, 8, D), jnp.float32),
            pltpu.VMEM((CHUNK, 8, D), jnp.float32),
            pltpu.VMEM((8, 128), jnp.float32),
            pltpu.VMEM((8, 128), jnp.float32),
            pltpu.VMEM((8, 128), jnp.float32),
            pltpu.VMEM((8, 128), jnp.float32),
            pltpu.SemaphoreType.DMA,
            pltpu.SemaphoreType.DMA,
            pltpu.SemaphoreType.DMA,
            pltpu.SemaphoreType.DMA,
            pltpu.SemaphoreType.DMA,
            pltpu.SemaphoreType.DMA,
            pltpu.SemaphoreType.DMA,
            pltpu.SemaphoreType.DMA,
        ],
        compiler_params=pltpu.CompilerParams(needs_layout_passes=False),
    )
    return lookup(x.reshape(BF), weight.reshape(weight.shape[0] // 8, 8, D))
